# baseline (speedup 1.0000x reference)
import jax
import jax.numpy as jnp
from jax import lax
from jax.experimental import pallas as pl
from jax.experimental.pallas import tpu as pltpu
from jax.experimental.pallas import tpu_sc as plsc

N = 10000
E = 160000
D = 256
HALF = 128
L = 16
NS = 16
EPT = E // NS
CH = 80
NCHUNK = EPT // CH
FL = 624
TAIL = N - NS * FL
NB = 3
CCH = 40
CEPT = E // 2 // NS
CNCHUNK = CEPT // CCH

_sc_mesh = plsc.VectorSubcoreMesh(core_axis_name="c", subcore_axis_name="s")


def _zero_fill(buf, nrows):
    zeros16 = jnp.zeros((L,), jnp.float32)

    def zf(i, _):
        buf[i // (HALF // L), pl.ds((i % (HALF // L)) * L, L)] = zeros16
        return 0
    lax.fori_loop(0, nrows * (HALF // L), zf, 0)


def _zero_acc_slice(zbuf, nb, acc, s):
    b0 = s * FL
    for k in range(FL // nb):
        pltpu.sync_copy(zbuf, acc.at[pl.ds(b0 + k * nb, nb)])
    rem = FL % nb
    if rem:
        pltpu.sync_copy(zbuf.at[pl.ds(0, rem)],
                        acc.at[pl.ds(b0 + (FL // nb) * nb, rem)])

    @pl.when(s == NS - 1)
    def _():
        pltpu.sync_copy(zbuf.at[pl.ds(0, TAIL)], acc.at[pl.ds(NS * FL, TAIL)])


def _flush_acc(acc, out_hbm, c, s):
    pltpu.sync_copy(acc.at[pl.ds(s * FL, FL)],
                    out_hbm.at[c, pl.ds(s * FL, FL)])

    @pl.when(s == NS - 1)
    def _():
        pltpu.sync_copy(acc.at[pl.ds(NS * FL, TAIL)],
                        out_hbm.at[c, pl.ds(NS * FL, TAIL)])


def _cnt_body(dst_hbm, cntp_hbm, dstv, onesb, cacc, dsem, ssem):
    c = lax.axis_index("c")
    s = lax.axis_index("s")
    _zero_fill(onesb, CCH)
    _zero_acc_slice(onesb, CCH, cacc, s)
    plsc.subcore_barrier()
    ones16 = jnp.ones((L,), jnp.float32)

    def of(i, _):
        onesb[i // (HALF // L), pl.ds((i % (HALF // L)) * L, L)] = ones16
        return 0
    lax.fori_loop(0, CCH * (HALF // L), of, 0)

    ebase = c * (E // 2) + s * CEPT

    def issue(i, b):
        pltpu.async_copy(dst_hbm.at[pl.ds(ebase + i * CCH, CCH)],
                         dstv.at[b], dsem.at[b])

    def wait_d(i, b):
        pltpu.make_async_copy(dst_hbm.at[pl.ds(ebase + i * CCH, CCH)],
                              dstv.at[b], dsem.at[b]).wait()

    def wait_s(b):
        pltpu.make_async_copy(onesb, cacc.at[dstv.at[b]],
                              ssem.at[b]).wait()

    for b in range(2):
        issue(b, b)

    def step(g, _):
        for b in range(NB):
            i = 3 * g + b
            wait_d(i, b)
            pltpu.async_copy(onesb, cacc.at[dstv.at[b]],
                             ssem.at[b], add=True)
            j = i + 2
            bj = (b + 2) % NB

            @pl.when(j >= NB)
            def _():
                wait_s(bj)
                issue(j, bj)

            @pl.when(j < NB)
            def _():
                issue(j, bj)
        return 0
    lax.fori_loop(0, (CNCHUNK - 2) // NB, step, 0)

    for i in range(CNCHUNK - 2, CNCHUNK):
        b = i % NB
        wait_d(i, b)
        pltpu.async_copy(onesb, cacc.at[dstv.at[b]],
                         ssem.at[b], add=True)
    for b in range(NB):
        wait_s(b)
    plsc.subcore_barrier()
    _flush_acc(cacc, cntp_hbm, c, s)


_cnt_kernel = pl.kernel(
    _cnt_body,
    out_type=[jax.ShapeDtypeStruct((2, N, HALF), jnp.float32)],
    mesh=_sc_mesh,
    scratch_types=[
        pltpu.VMEM((NB, CCH), jnp.int32),
        pltpu.VMEM((CCH, HALF), jnp.float32),
        pltpu.VMEM_SHARED((N, HALF), jnp.float32),
        pltpu.SemaphoreType.DMA((NB,)),
        pltpu.SemaphoreType.DMA((NB,)),
    ],
)


NB = 3


def _seg_sum_body(x_hbm, src_hbm, dst_hbm, summed_hbm,
                  srcall, dstv, rows, acc, dsem, gsem, ssem):
    c = lax.axis_index("c")
    s = lax.axis_index("s")
    ebase = s * EPT
    off = c * N

    pltpu.sync_copy(src_hbm.at[pl.ds(ebase, EPT)], srcall)

    def adj(k, _):
        srcall[pl.ds(k * L, L)] = srcall[pl.ds(k * L, L)] + off
        return 0
    lax.fori_loop(0, EPT // L, adj, 0)

    _zero_fill(rows.at[0], CH)
    _zero_acc_slice(rows.at[0], CH, acc, s)
    plsc.subcore_barrier()

    def issue(i, b):
        pltpu.async_copy(dst_hbm.at[pl.ds(ebase + i * CH, CH)],
                         dstv.at[b], dsem.at[b])
        pltpu.async_copy(x_hbm.at[srcall.at[pl.ds(i * CH, CH)]],
                         rows.at[b], gsem.at[b])

    def wait_dg(i, b):
        pltpu.make_async_copy(dst_hbm.at[pl.ds(ebase + i * CH, CH)],
                              dstv.at[b], dsem.at[b]).wait()
        pltpu.make_async_copy(x_hbm.at[srcall.at[pl.ds(i * CH, CH)]],
                              rows.at[b], gsem.at[b]).wait()

    def wait_s(b):
        pltpu.make_async_copy(rows.at[b], acc.at[dstv.at[b]],
                              ssem.at[b]).wait()

    for b in range(2):
        issue(b, b)

    def step(g, _):
        for b in range(NB):
            i = 3 * g + b
            wait_dg(i, b)
            pltpu.async_copy(rows.at[b], acc.at[dstv.at[b]],
                             ssem.at[b], add=True)
            j = i + 2
            bj = (b + 2) % NB

            @pl.when(j >= NB)
            def _():
                wait_s(bj)
                issue(j, bj)

            @pl.when(j < NB)
            def _():
                issue(j, bj)
        return 0
    lax.fori_loop(0, (NCHUNK - 2) // NB, step, 0)

    for i in range(NCHUNK - 2, NCHUNK):
        b = i % NB
        wait_dg(i, b)
        pltpu.async_copy(rows.at[b], acc.at[dstv.at[b]],
                         ssem.at[b], add=True)
    for b in range(NB):
        wait_s(b)
    plsc.subcore_barrier()
    _flush_acc(acc, summed_hbm, c, s)


_seg_sum = pl.kernel(
    _seg_sum_body,
    out_type=[jax.ShapeDtypeStruct((2, N, HALF), jnp.float32)],
    mesh=_sc_mesh,
    scratch_types=[
        pltpu.VMEM((EPT,), jnp.int32),
        pltpu.VMEM((NB, CH), jnp.int32),
        pltpu.VMEM((NB, CH, HALF), jnp.float32),
        pltpu.VMEM_SHARED((N, HALF), jnp.float32),
        pltpu.SemaphoreType.DMA((NB,)),
        pltpu.SemaphoreType.DMA((NB,)),
        pltpu.SemaphoreType.DMA((NB,)),
    ],
)


GB = 4
GCH2 = 40
GEPT = E // 2 // NS
GNCH = GEPT // GCH2
GV = 2 * GNCH


def _edge_gather_body(h_hbm, src_hbm, dst_hbm, hr_hbm, hc_hbm,
                      srcall, dstall, rows, gsem, wsem):
    c = lax.axis_index("c")
    s = lax.axis_index("s")
    ebase = c * (E // 2) + s * GEPT

    pltpu.sync_copy(src_hbm.at[pl.ds(ebase, GEPT)], srcall)
    pltpu.sync_copy(dst_hbm.at[pl.ds(ebase, GEPT)], dstall)

    def gather_issue(v, b):
        @pl.when(v < GNCH)
        def _():
            pltpu.async_copy(h_hbm.at[srcall.at[pl.ds(v * GCH2, GCH2)]],
                             rows.at[b], gsem.at[b])

        @pl.when(v >= GNCH)
        def _():
            pltpu.async_copy(h_hbm.at[dstall.at[pl.ds((v - GNCH) * GCH2, GCH2)]],
                             rows.at[b], gsem.at[b])

    def gather_wait(v, b):
        pltpu.make_async_copy(h_hbm.at[srcall.at[pl.ds(0, GCH2)]],
                              rows.at[b], gsem.at[b]).wait()

    def write_issue(v, b):
        @pl.when(v < GNCH)
        def _():
            pltpu.async_copy(rows.at[b],
                             hr_hbm.at[pl.ds(ebase + v * GCH2, GCH2)],
                             wsem.at[b])

        @pl.when(v >= GNCH)
        def _():
            pltpu.async_copy(rows.at[b],
                             hc_hbm.at[pl.ds(ebase + (v - GNCH) * GCH2, GCH2)],
                             wsem.at[b])

    def write_wait(b):
        pltpu.make_async_copy(rows.at[b],
                              hr_hbm.at[pl.ds(ebase, GCH2)],
                              wsem.at[b]).wait()

    for b in range(3):
        gather_issue(jnp.int32(b), b)

    def step(g, _):
        for b in range(GB):
            v = GB * g + b
            gather_wait(v, b)
            write_issue(v, b)
            j = v + 3
            bj = (b + 3) % GB

            @pl.when((j >= GB) & (j < GV))
            def _():
                write_wait(bj)
                gather_issue(j, bj)

            @pl.when(j < GB)
            def _():
                gather_issue(jnp.int32(j), bj)
        return 0
    lax.fori_loop(0, (GV - 3) // GB, step, 0)

    for v in range(((GV - 3) // GB) * GB, GV):
        b = v % GB
        gather_wait(v, b)
        write_issue(jnp.int32(v), b)
        j = v + 3
        if j < GV:
            bj = (b + 3) % GB
            write_wait(bj)
            gather_issue(jnp.int32(j), bj)
    for b in range(GB):
        write_wait(b)


_edge_gather = pl.kernel(
    _edge_gather_body,
    out_type=[jax.ShapeDtypeStruct((E, HALF), jnp.int32),
              jax.ShapeDtypeStruct((E, HALF), jnp.int32)],
    mesh=_sc_mesh,
    scratch_types=[
        pltpu.VMEM((GEPT,), jnp.int32),
        pltpu.VMEM((GEPT,), jnp.int32),
        pltpu.VMEM((GB, GCH2, HALF), jnp.int32),
        pltpu.SemaphoreType.DMA((GB,)),
        pltpu.SemaphoreType.DMA((GB,)),
    ],
)


def _sage_body(s_ref, c_ref, p_ref, wl_ref, bl_ref, wr_ref, o_ref):
    inv = 1.0 / jnp.maximum(c_ref[...], 1.0)
    wl = wl_ref[...]
    wr = wr_ref[...]
    t = (jnp.dot(s_ref[0] * inv, wl[:HALF], preferred_element_type=jnp.float32)
         + jnp.dot(s_ref[1] * inv, wl[HALF:], preferred_element_type=jnp.float32)
         + jnp.dot(p_ref[0], wr[:HALF], preferred_element_type=jnp.float32)
         + jnp.dot(p_ref[1], wr[HALF:], preferred_element_type=jnp.float32)
         + bl_ref[...])
    h = jnp.maximum(t, 0.0)
    if o_ref.shape[0] == 2:
        o_ref[0] = h[:, :HALF].astype(o_ref.dtype)
        o_ref[1] = h[:, HALF:].astype(o_ref.dtype)
    else:
        o_ref[...] = h.astype(o_ref.dtype)


def _sage_tc(summed, cnt, prev, WlT, bl, WrT, out_dtype=jnp.float32,
             flat=False):
    BN = 2000
    if flat:
        out_spec = pl.BlockSpec((BN, D), lambda i: (i, 0))
        out_shape = jax.ShapeDtypeStruct((N, D), out_dtype)
    else:
        out_spec = pl.BlockSpec((2, BN, HALF), lambda i: (0, i, 0))
        out_shape = jax.ShapeDtypeStruct((2, N, HALF), out_dtype)
    return pl.pallas_call(
        _sage_body,
        grid=(N // BN,),
        in_specs=[
            pl.BlockSpec((2, BN, HALF), lambda i: (0, i, 0)),
            pl.BlockSpec((BN, 1), lambda i: (i, 0)),
            pl.BlockSpec((2, BN, HALF), lambda i: (0, i, 0)),
            pl.BlockSpec((D, D), lambda i: (0, 0)),
            pl.BlockSpec((1, D), lambda i: (0, 0)),
            pl.BlockSpec((D, D), lambda i: (0, 0)),
        ],
        out_specs=out_spec,
        out_shape=out_shape,
    )(summed, cnt, prev, WlT, bl, WrT)


def _edge_mlp_body(r_ref, c_ref, w1_ref, b1_ref, w2_ref, b2_ref, o_ref):
    r = r_ref[...]
    cc = c_ref[...]
    w1 = w1_ref[...]
    z = (jnp.dot(jnp.abs(r - cc), w1[:D],
                 preferred_element_type=jnp.float32)
         + jnp.dot(r * cc, w1[D:],
                   preferred_element_type=jnp.float32)
         + b1_ref[...])
    z = jnp.maximum(z, 0.0)
    u = jnp.sum(z * w2_ref[...], axis=1) + b2_ref[0, 0]
    i = pl.program_id(0)
    o_ref[pl.ds(i * u.shape[0], u.shape[0])] = (
        jnp.maximum(u, 0.0) + jnp.log1p(jnp.exp(-jnp.abs(u))))


def _edge_mlp_tc(hr, hc, Wp1T, bp1, Wp2, bp2):
    BE = 1280
    return pl.pallas_call(
        _edge_mlp_body,
        grid=(E // BE,),
        in_specs=[
            pl.BlockSpec((BE, D), lambda i: (i, 0)),
            pl.BlockSpec((BE, D), lambda i: (i, 0)),
            pl.BlockSpec((2 * D, D), lambda i: (0, 0)),
            pl.BlockSpec((1, D), lambda i: (0, 0)),
            pl.BlockSpec((1, D), lambda i: (0, 0)),
            pl.BlockSpec((1, 1), lambda i: (0, 0)),
        ],
        out_specs=pl.BlockSpec((E,), lambda i: (0,)),
        out_shape=jax.ShapeDtypeStruct((E,), jnp.float32),
    )(hr, hc, Wp1T, bp1, Wp2, bp2)


def kernel(x, edge_index, W_l0, b_l0, W_r0, W_l1, b_l1, W_r1, Wp1, bp1, Wp2, bp2):
    src = edge_index[0].astype(jnp.int32)
    dst = edge_index[1].astype(jnp.int32)
    xh = x.reshape(N, 2, HALF).transpose(1, 0, 2)
    cntp = _cnt_kernel(dst)[0]
    cnt = cntp[0, :, 0:1] + cntp[1, :, 0:1]
    summed1 = _seg_sum(xh.reshape(2 * N, HALF), src, dst)[0]
    h1 = _sage_tc(summed1, cnt, xh, W_l0.T, b_l0.reshape(1, D), W_r0.T)
    summed2 = _seg_sum(h1.reshape(2 * N, HALF), src, dst)[0]
    h2 = _sage_tc(summed2, cnt, h1, W_l1.T, b_l1.reshape(1, D), W_r1.T,
                  out_dtype=jnp.bfloat16, flat=True)
    h2w = lax.bitcast_convert_type(
        h2.reshape(N, HALF, 2), jnp.int32)
    hrw, hcw = _edge_gather(h2w, src, dst)
    hr = lax.bitcast_convert_type(hrw, jnp.bfloat16).reshape(E, D)
    hc = lax.bitcast_convert_type(hcw, jnp.bfloat16).reshape(E, D)
    return _edge_mlp_tc(hr, hc, Wp1.T.astype(jnp.bfloat16), bp1.reshape(1, D),
                        Wp2, bp2.reshape(1, 1))


# speedup vs baseline: 2.7403x; 2.7403x over previous
import jax
import jax.numpy as jnp
from jax import lax
from jax.experimental import pallas as pl
from jax.experimental.pallas import tpu as pltpu
from jax.experimental.pallas import tpu_sc as plsc

N = 10000
E = 160000
D = 256
HALF = 128
L = 16
NS = 16
EPT = E // NS
CH = 80
NCHUNK = EPT // CH
FL = 624
TAIL = N - NS * FL
NB = 3
CCH = 40
CEPT = E // 2 // NS
CNCHUNK = CEPT // CCH

_sc_mesh = plsc.VectorSubcoreMesh(core_axis_name="c", subcore_axis_name="s")


def _zero_fill(buf, nrows):
    zeros16 = jnp.zeros((L,), jnp.float32)

    def zf(i, _):
        buf[i // (HALF // L), pl.ds((i % (HALF // L)) * L, L)] = zeros16
        return 0
    lax.fori_loop(0, nrows * (HALF // L), zf, 0)


def _zero_acc_slice(zbuf, nb, acc, s):
    b0 = s * FL
    for k in range(FL // nb):
        pltpu.sync_copy(zbuf, acc.at[pl.ds(b0 + k * nb, nb)])
    rem = FL % nb
    if rem:
        pltpu.sync_copy(zbuf.at[pl.ds(0, rem)],
                        acc.at[pl.ds(b0 + (FL // nb) * nb, rem)])

    @pl.when(s == NS - 1)
    def _():
        pltpu.sync_copy(zbuf.at[pl.ds(0, TAIL)], acc.at[pl.ds(NS * FL, TAIL)])


def _flush_acc(acc, out_hbm, c, s):
    pltpu.sync_copy(acc.at[pl.ds(s * FL, FL)],
                    out_hbm.at[c, pl.ds(s * FL, FL)])

    @pl.when(s == NS - 1)
    def _():
        pltpu.sync_copy(acc.at[pl.ds(NS * FL, TAIL)],
                        out_hbm.at[c, pl.ds(NS * FL, TAIL)])


def _cnt_body(dst_hbm, cntp_hbm, dstv, onesb, cacc, dsem, ssem):
    c = lax.axis_index("c")
    s = lax.axis_index("s")
    _zero_fill(onesb, CCH)
    _zero_acc_slice(onesb, CCH, cacc, s)
    plsc.subcore_barrier()
    ones16 = jnp.ones((L,), jnp.float32)

    def of(i, _):
        onesb[i // (HALF // L), pl.ds((i % (HALF // L)) * L, L)] = ones16
        return 0
    lax.fori_loop(0, CCH * (HALF // L), of, 0)

    ebase = c * (E // 2) + s * CEPT

    def issue(i, b):
        pltpu.async_copy(dst_hbm.at[pl.ds(ebase + i * CCH, CCH)],
                         dstv.at[b], dsem.at[b])

    def wait_d(i, b):
        pltpu.make_async_copy(dst_hbm.at[pl.ds(ebase + i * CCH, CCH)],
                              dstv.at[b], dsem.at[b]).wait()

    def wait_s(b):
        pltpu.make_async_copy(onesb, cacc.at[dstv.at[b]],
                              ssem.at[b]).wait()

    for b in range(2):
        issue(b, b)

    def step(g, _):
        for b in range(NB):
            i = 3 * g + b
            wait_d(i, b)
            pltpu.async_copy(onesb, cacc.at[dstv.at[b]],
                             ssem.at[b], add=True)
            j = i + 2
            bj = (b + 2) % NB

            @pl.when(j >= NB)
            def _():
                wait_s(bj)
                issue(j, bj)

            @pl.when(j < NB)
            def _():
                issue(j, bj)
        return 0
    lax.fori_loop(0, (CNCHUNK - 2) // NB, step, 0)

    for i in range(CNCHUNK - 2, CNCHUNK):
        b = i % NB
        wait_d(i, b)
        pltpu.async_copy(onesb, cacc.at[dstv.at[b]],
                         ssem.at[b], add=True)
    for b in range(NB):
        wait_s(b)
    plsc.subcore_barrier()
    _flush_acc(cacc, cntp_hbm, c, s)


_cnt_kernel = pl.kernel(
    _cnt_body,
    out_type=[jax.ShapeDtypeStruct((2, N, HALF), jnp.float32)],
    mesh=_sc_mesh,
    scratch_types=[
        pltpu.VMEM((NB, CCH), jnp.int32),
        pltpu.VMEM((CCH, HALF), jnp.float32),
        pltpu.VMEM_SHARED((N, HALF), jnp.float32),
        pltpu.SemaphoreType.DMA((NB,)),
        pltpu.SemaphoreType.DMA((NB,)),
    ],
)


NB = 3


def _seg_sum_body(x_hbm, src_hbm, dst_hbm, summed_hbm,
                  srcall, dstv, rows, acc, dsem, gsem, ssem):
    c = lax.axis_index("c")
    s = lax.axis_index("s")
    ebase = s * EPT
    off = c * N

    pltpu.sync_copy(src_hbm.at[pl.ds(ebase, EPT)], srcall)

    def adj(k, _):
        srcall[pl.ds(k * L, L)] = srcall[pl.ds(k * L, L)] + off
        return 0
    lax.fori_loop(0, EPT // L, adj, 0)

    _zero_fill(rows.at[0], CH)
    _zero_acc_slice(rows.at[0], CH, acc, s)
    plsc.subcore_barrier()

    def issue(i, b):
        pltpu.async_copy(dst_hbm.at[pl.ds(ebase + i * CH, CH)],
                         dstv.at[b], dsem.at[b])
        pltpu.async_copy(x_hbm.at[srcall.at[pl.ds(i * CH, CH)]],
                         rows.at[b], gsem.at[b])

    def wait_dg(i, b):
        pltpu.make_async_copy(dst_hbm.at[pl.ds(ebase + i * CH, CH)],
                              dstv.at[b], dsem.at[b]).wait()
        pltpu.make_async_copy(x_hbm.at[srcall.at[pl.ds(i * CH, CH)]],
                              rows.at[b], gsem.at[b]).wait()

    def wait_s(b):
        pltpu.make_async_copy(rows.at[b], acc.at[dstv.at[b]],
                              ssem.at[b]).wait()

    for b in range(2):
        issue(b, b)

    def step(g, _):
        for b in range(NB):
            i = 3 * g + b
            wait_dg(i, b)
            pltpu.async_copy(rows.at[b], acc.at[dstv.at[b]],
                             ssem.at[b], add=True)
            j = i + 2
            bj = (b + 2) % NB

            @pl.when(j >= NB)
            def _():
                wait_s(bj)
                issue(j, bj)

            @pl.when(j < NB)
            def _():
                issue(j, bj)
        return 0
    lax.fori_loop(0, (NCHUNK - 2) // NB, step, 0)

    for i in range(NCHUNK - 2, NCHUNK):
        b = i % NB
        wait_dg(i, b)
        pltpu.async_copy(rows.at[b], acc.at[dstv.at[b]],
                         ssem.at[b], add=True)
    for b in range(NB):
        wait_s(b)
    plsc.subcore_barrier()
    _flush_acc(acc, summed_hbm, c, s)


_seg_sum = pl.kernel(
    _seg_sum_body,
    out_type=[jax.ShapeDtypeStruct((2, N, HALF), jnp.float32)],
    mesh=_sc_mesh,
    scratch_types=[
        pltpu.VMEM((EPT,), jnp.int32),
        pltpu.VMEM((NB, CH), jnp.int32),
        pltpu.VMEM((NB, CH, HALF), jnp.float32),
        pltpu.VMEM_SHARED((N, HALF), jnp.float32),
        pltpu.SemaphoreType.DMA((NB,)),
        pltpu.SemaphoreType.DMA((NB,)),
        pltpu.SemaphoreType.DMA((NB,)),
    ],
)


GB = 4
GCH2 = 40
GEPT = E // 2 // NS
GNCH = GEPT // GCH2
GV = 2 * GNCH


def _edge_gather_body(h_hbm, src_hbm, dst_hbm, hr_hbm, hc_hbm,
                      srcall, dstall, rows, gsem, wsem):
    c = lax.axis_index("c")
    s = lax.axis_index("s")
    ebase = c * (E // 2) + s * GEPT

    pltpu.sync_copy(src_hbm.at[pl.ds(ebase, GEPT)], srcall)
    pltpu.sync_copy(dst_hbm.at[pl.ds(ebase, GEPT)], dstall)

    def gather_issue(v, b):
        @pl.when(v < GNCH)
        def _():
            pltpu.async_copy(h_hbm.at[srcall.at[pl.ds(v * GCH2, GCH2)]],
                             rows.at[b], gsem.at[b])

        @pl.when(v >= GNCH)
        def _():
            pltpu.async_copy(h_hbm.at[dstall.at[pl.ds((v - GNCH) * GCH2, GCH2)]],
                             rows.at[b], gsem.at[b])

    def gather_wait(v, b):
        pltpu.make_async_copy(h_hbm.at[srcall.at[pl.ds(0, GCH2)]],
                              rows.at[b], gsem.at[b]).wait()

    def write_issue(v, b):
        @pl.when(v < GNCH)
        def _():
            pltpu.async_copy(rows.at[b],
                             hr_hbm.at[pl.ds(ebase + v * GCH2, GCH2)],
                             wsem.at[b])

        @pl.when(v >= GNCH)
        def _():
            pltpu.async_copy(rows.at[b],
                             hc_hbm.at[pl.ds(ebase + (v - GNCH) * GCH2, GCH2)],
                             wsem.at[b])

    def write_wait(b):
        pltpu.make_async_copy(rows.at[b],
                              hr_hbm.at[pl.ds(ebase, GCH2)],
                              wsem.at[b]).wait()

    for b in range(3):
        gather_issue(jnp.int32(b), b)

    def step(g, _):
        for b in range(GB):
            v = GB * g + b
            gather_wait(v, b)
            write_issue(v, b)
            j = v + 3
            bj = (b + 3) % GB

            @pl.when((j >= GB) & (j < GV))
            def _():
                write_wait(bj)
                gather_issue(j, bj)

            @pl.when(j < GB)
            def _():
                gather_issue(jnp.int32(j), bj)
        return 0
    lax.fori_loop(0, (GV - 3) // GB, step, 0)

    for v in range(((GV - 3) // GB) * GB, GV):
        b = v % GB
        gather_wait(v, b)
        write_issue(jnp.int32(v), b)
        j = v + 3
        if j < GV:
            bj = (b + 3) % GB
            write_wait(bj)
            gather_issue(jnp.int32(j), bj)
    for b in range(GB):
        write_wait(b)


_edge_gather = pl.kernel(
    _edge_gather_body,
    out_type=[jax.ShapeDtypeStruct((E, HALF), jnp.int32),
              jax.ShapeDtypeStruct((E, HALF), jnp.int32)],
    mesh=_sc_mesh,
    scratch_types=[
        pltpu.VMEM((GEPT,), jnp.int32),
        pltpu.VMEM((GEPT,), jnp.int32),
        pltpu.VMEM((GB, GCH2, HALF), jnp.int32),
        pltpu.SemaphoreType.DMA((GB,)),
        pltpu.SemaphoreType.DMA((GB,)),
    ],
)


def _sage_body(s_ref, c_ref, p_ref, wl_ref, bl_ref, wr_ref, o_ref):
    inv = 1.0 / jnp.maximum(c_ref[...], 1.0)
    wl = wl_ref[...]
    wr = wr_ref[...]
    t = (jnp.dot(s_ref[0] * inv, wl[:HALF], preferred_element_type=jnp.float32)
         + jnp.dot(s_ref[1] * inv, wl[HALF:], preferred_element_type=jnp.float32)
         + jnp.dot(p_ref[0], wr[:HALF], preferred_element_type=jnp.float32)
         + jnp.dot(p_ref[1], wr[HALF:], preferred_element_type=jnp.float32)
         + bl_ref[...])
    h = jnp.maximum(t, 0.0)
    if o_ref.shape[0] == 2:
        o_ref[0] = h[:, :HALF]
        o_ref[1] = h[:, HALF:]
    else:
        lo = lax.bitcast_convert_type(
            h[:, :HALF].astype(jnp.bfloat16), jnp.uint16).astype(jnp.int32)
        hi = lax.bitcast_convert_type(
            h[:, HALF:].astype(jnp.bfloat16), jnp.uint16).astype(jnp.int32)
        o_ref[...] = lo | (hi << 16)


def _sage_tc(summed, cnt, prev, WlT, bl, WrT, packed=False):
    BN = 2000
    if packed:
        out_spec = pl.BlockSpec((BN, HALF), lambda i: (i, 0))
        out_shape = jax.ShapeDtypeStruct((N, HALF), jnp.int32)
    else:
        out_spec = pl.BlockSpec((2, BN, HALF), lambda i: (0, i, 0))
        out_shape = jax.ShapeDtypeStruct((2, N, HALF), jnp.float32)
    return pl.pallas_call(
        _sage_body,
        grid=(N // BN,),
        in_specs=[
            pl.BlockSpec((2, BN, HALF), lambda i: (0, i, 0)),
            pl.BlockSpec((BN, 1), lambda i: (i, 0)),
            pl.BlockSpec((2, BN, HALF), lambda i: (0, i, 0)),
            pl.BlockSpec((D, D), lambda i: (0, 0)),
            pl.BlockSpec((1, D), lambda i: (0, 0)),
            pl.BlockSpec((D, D), lambda i: (0, 0)),
        ],
        out_specs=out_spec,
        out_shape=out_shape,
    )(summed, cnt, prev, WlT, bl, WrT)


def _unpack_bf16_pair(w):
    lo = lax.bitcast_convert_type(w << 16, jnp.float32)
    hi = lax.bitcast_convert_type(w & jnp.int32(-65536), jnp.float32)
    return lo, hi


def _edge_mlp_body(r_ref, c_ref, w1_ref, b1_ref, w2_ref, b2_ref, o_ref):
    r0, r1 = _unpack_bf16_pair(r_ref[...])
    c0, c1 = _unpack_bf16_pair(c_ref[...])
    w1 = w1_ref[...]
    bf = jnp.bfloat16
    z = (jnp.dot(jnp.abs(r0 - c0).astype(bf), w1[0:HALF],
                 preferred_element_type=jnp.float32)
         + jnp.dot(jnp.abs(r1 - c1).astype(bf), w1[HALF:2 * HALF],
                   preferred_element_type=jnp.float32)
         + jnp.dot((r0 * c0).astype(bf), w1[2 * HALF:3 * HALF],
                   preferred_element_type=jnp.float32)
         + jnp.dot((r1 * c1).astype(bf), w1[3 * HALF:],
                   preferred_element_type=jnp.float32)
         + b1_ref[...])
    z = jnp.maximum(z, 0.0)
    u = jnp.sum(z * w2_ref[...], axis=1) + b2_ref[0, 0]
    i = pl.program_id(0)
    o_ref[pl.ds(i * u.shape[0], u.shape[0])] = (
        jnp.maximum(u, 0.0) + jnp.log1p(jnp.exp(-jnp.abs(u))))


def _edge_mlp_tc(hr, hc, Wp1T, bp1, Wp2, bp2):
    BE = 1280
    return pl.pallas_call(
        _edge_mlp_body,
        grid=(E // BE,),
        in_specs=[
            pl.BlockSpec((BE, HALF), lambda i: (i, 0)),
            pl.BlockSpec((BE, HALF), lambda i: (i, 0)),
            pl.BlockSpec((2 * D, D), lambda i: (0, 0)),
            pl.BlockSpec((1, D), lambda i: (0, 0)),
            pl.BlockSpec((1, D), lambda i: (0, 0)),
            pl.BlockSpec((1, 1), lambda i: (0, 0)),
        ],
        out_specs=pl.BlockSpec((E,), lambda i: (0,)),
        out_shape=jax.ShapeDtypeStruct((E,), jnp.float32),
    )(hr, hc, Wp1T, bp1, Wp2, bp2)


def kernel(x, edge_index, W_l0, b_l0, W_r0, W_l1, b_l1, W_r1, Wp1, bp1, Wp2, bp2):
    src = edge_index[0].astype(jnp.int32)
    dst = edge_index[1].astype(jnp.int32)
    xh = x.reshape(N, 2, HALF).transpose(1, 0, 2)
    cntp = _cnt_kernel(dst)[0]
    cnt = cntp[0, :, 0:1] + cntp[1, :, 0:1]
    summed1 = _seg_sum(xh.reshape(2 * N, HALF), src, dst)[0]
    h1 = _sage_tc(summed1, cnt, xh, W_l0.T, b_l0.reshape(1, D), W_r0.T)
    summed2 = _seg_sum(h1.reshape(2 * N, HALF), src, dst)[0]
    h2w = _sage_tc(summed2, cnt, h1, W_l1.T, b_l1.reshape(1, D), W_r1.T,
                   packed=True)
    hrw, hcw = _edge_gather(h2w, src, dst)
    return _edge_mlp_tc(hrw, hcw, Wp1.T.astype(jnp.bfloat16),
                        bp1.reshape(1, D), Wp2, bp2.reshape(1, 1))
